# baseline (speedup 1.0000x reference)
import functools

import jax
import jax.numpy as jnp
from jax import lax
from jax.experimental import pallas as pl
from jax.experimental.pallas import tpu as pltpu
from jax.experimental.pallas import tpu_sc as plsc


def _matmul_body(w_ref, xt_ref, o_ref):
    o_ref[...] = jnp.dot(w_ref[...], xt_ref[...],
                         preferred_element_type=jnp.float32)


def _matmul_flat_body(w_ref, xt_ref, o_ref):
    k = pl.program_id(1)
    o_ref[...] = jnp.dot(w_ref[pl.ds(k, 1), :], xt_ref[...],
                         preferred_element_type=jnp.float32)


def _gate_body(xt_ref, a_ref, b_ref, o_ref):
    att = jax.nn.sigmoid(a_ref[...] + b_ref[...])
    o_ref[...] = xt_ref[...] * att


def _make_sc_gather(K, N, NP):
    f32 = jnp.float32
    PT = NP // 16
    ROWS = 14
    U = 8

    mesh = plsc.VectorSubcoreMesh(core_axis_name="c", subcore_axis_name="s")

    @functools.partial(
        pl.kernel,
        out_type=[jax.ShapeDtypeStruct((NP,), f32),
                  jax.ShapeDtypeStruct((NP,), f32)],
        mesh=mesh,
        compiler_params=pltpu.CompilerParams(needs_layout_passes=False),
        scratch_types=[
            pltpu.VMEM((PT,), jnp.int32),
            pltpu.VMEM((PT,), jnp.int32),
            pltpu.VMEM((PT,), f32),
            pltpu.VMEM((PT,), f32),
            pltpu.VMEM((PT,), f32),
            pltpu.VMEM_SHARED((ROWS * NP,), f32),
            pltpu.SemaphoreType.DMA,
            pltpu.SemaphoreType.DMA,
        ],
    )
    def sc_gather(gidx_hbm, yt_hbm, s0_hbm, s1_hbm,
                  idx0, idx1, gb0, gb1, acc, ytsh, sem_i, sem_g):
        c = lax.axis_index("c")
        s = lax.axis_index("s")

        n_real = 14 - c

        @pl.when(s < n_real)
        def _stage():
            krow = c * ROWS + s
            nfull = NP // PT
            bufs = (gb0, gb1)
            h = pltpu.async_copy(yt_hbm.at[pl.ds(krow * NP, PT)], gb0, sem_i)
            for ci in range(nfull):
                h.wait()
                if ci + 1 < nfull:
                    h = pltpu.async_copy(
                        yt_hbm.at[pl.ds(krow * NP + (ci + 1) * PT, PT)],
                        bufs[(ci + 1) % 2], sem_i)
                pltpu.sync_copy(bufs[ci % 2],
                                ytsh.at[pl.ds(s * NP + ci * PT, PT)])

        @pl.when((c == 1) & (s == ROWS - 1))
        def _zero_pad_row():
            def zv(j, carry):
                o = j * (16 * U)
                for u in range(U):
                    gb0[pl.ds(o + u * 16, 16)] = jnp.zeros((16,), f32)
                return carry

            lax.fori_loop(0, PT // (16 * U), zv, 0)
            for ci in range(NP // PT):
                pltpu.sync_copy(
                    gb0, ytsh.at[pl.ds((ROWS - 1) * NP + ci * PT, PT)])

        plsc.subcore_barrier()

        rbase = c * ROWS * NP + s * PT
        idxb = (idx0, idx1)
        gbufs = (gb0, gb1)

        pltpu.async_copy(gidx_hbm.at[pl.ds(rbase, PT)], idx0, sem_i).wait()
        gathers = [None] * ROWS
        gathers[0] = pltpu.async_copy(ytsh.at[idx0], gb0, sem_g)
        idx_pending = pltpu.async_copy(
            gidx_hbm.at[pl.ds(rbase + NP, PT)], idx1, sem_i)

        for g in range(ROWS):
            gathers[g].wait()
            if g + 1 < ROWS:
                idx_pending.wait()
                gathers[g + 1] = pltpu.async_copy(
                    ytsh.at[idxb[(g + 1) % 2]], gbufs[(g + 1) % 2], sem_g)
                if g + 2 < ROWS:
                    idx_pending = pltpu.async_copy(
                        gidx_hbm.at[pl.ds(rbase + (g + 2) * NP, PT)],
                        idxb[g % 2], sem_i)
            gb = gbufs[g % 2]

            def accum(j, carry, gb=gb, first=(g == 0)):
                o = j * (16 * U)
                for u in range(U):
                    oo = o + u * 16
                    if first:
                        acc[pl.ds(oo, 16)] = gb[pl.ds(oo, 16)]
                    else:
                        acc[pl.ds(oo, 16)] = acc[pl.ds(oo, 16)] + gb[pl.ds(oo, 16)]
                return carry

            lax.fori_loop(0, PT // (16 * U), accum, 0)

        @pl.when(c == 0)
        def _w0():
            pltpu.sync_copy(acc, s0_hbm.at[pl.ds(s * PT, PT)])

        @pl.when(c == 1)
        def _w1():
            pltpu.sync_copy(acc, s1_hbm.at[pl.ds(s * PT, PT)])

    return sc_gather


def kernel(x, neighbor_map, W):
    N, C = x.shape
    K = neighbor_map.shape[1]
    f32 = jnp.float32

    BC = 4096
    BA = 2048
    NB = (N + BA - 1) // BA
    NP = NB * BA

    Wk = W.reshape(K, C)
    xT = x.T
    PT = NP // 16
    offs = jnp.concatenate([
        jnp.arange(14, dtype=jnp.int32),
        jnp.arange(13, dtype=jnp.int32),
        jnp.full((1,), 13, dtype=jnp.int32)]) * NP
    nmT28 = jnp.concatenate([
        neighbor_map.T.astype(jnp.int32),
        jnp.zeros((1, N), jnp.int32)], axis=0)
    gidx = jnp.pad(nmT28 + offs[:, None], ((0, 0), (0, NP - N))).reshape(-1)

    yt = pl.pallas_call(
        _matmul_flat_body,
        grid=(NB, K),
        in_specs=[pl.BlockSpec((K, C), lambda i, k: (0, 0)),
                  pl.BlockSpec((C, BA), lambda i, k: (0, i))],
        out_specs=pl.BlockSpec((1, BA), lambda i, k: (0, k * NB + i)),
        out_shape=jax.ShapeDtypeStruct((1, K * NP), f32),
    )(Wk, xT)

    sc = _make_sc_gather(K, N, NP)
    s0, s1 = sc(gidx, yt.reshape(-1))

    s0t = s0.reshape(1, NP)
    s1t = s1.reshape(1, NP)
    BB = 2048
    gb = (N + BB - 1) // BB
    outT = pl.pallas_call(
        _gate_body,
        grid=(gb,),
        in_specs=[pl.BlockSpec((C, BB), lambda i: (0, i)),
                  pl.BlockSpec((1, BB), lambda i: (0, i)),
                  pl.BlockSpec((1, BB), lambda i: (0, i))],
        out_specs=pl.BlockSpec((C, BB), lambda i: (0, i)),
        out_shape=jax.ShapeDtypeStruct((C, N), f32),
    )(xT, s0t, s1t)
    return outT.T


# speedup vs baseline: 3.0897x; 3.0897x over previous
import functools

import jax
import jax.numpy as jnp
from jax import lax
from jax.experimental import pallas as pl
from jax.experimental.pallas import tpu as pltpu
from jax.experimental.pallas import tpu_sc as plsc


def _matmul_body(w_ref, xt_ref, o_ref):
    o_ref[...] = jnp.dot(w_ref[...], xt_ref[...],
                         preferred_element_type=jnp.float32)


def _matmul_flat_body(w_ref, xt_ref, o_ref):
    k = pl.program_id(1)
    o_ref[...] = jnp.dot(w_ref[pl.ds(k, 1), :], xt_ref[...],
                         preferred_element_type=jnp.float32)


def _gate_body(xt_ref, a_ref, b_ref, o_ref):
    att = jax.nn.sigmoid(a_ref[...] + b_ref[...])
    o_ref[...] = xt_ref[...] * att


def _make_sc_gather(K, N, NP):
    f32 = jnp.float32
    PT = NP // 16
    ROWS = 14
    U = 8

    mesh = plsc.VectorSubcoreMesh(core_axis_name="c", subcore_axis_name="s")

    @functools.partial(
        pl.kernel,
        out_type=[jax.ShapeDtypeStruct((NP,), f32),
                  jax.ShapeDtypeStruct((NP,), f32)],
        mesh=mesh,
        compiler_params=pltpu.CompilerParams(needs_layout_passes=False),
        scratch_types=[
            pltpu.VMEM((PT,), jnp.int32),
            pltpu.VMEM((PT,), jnp.int32),
            pltpu.VMEM((PT,), f32),
            pltpu.VMEM((PT,), f32),
            pltpu.VMEM((PT,), f32),
            pltpu.VMEM_SHARED((ROWS * NP,), f32),
            pltpu.SemaphoreType.DMA,
            pltpu.SemaphoreType.DMA,
        ],
    )
    def sc_gather(gidx_hbm, yt_hbm, s0_hbm, s1_hbm,
                  idx0, idx1, gb0, gb1, acc, ytsh, sem_i, sem_g):
        c = lax.axis_index("c")
        s = lax.axis_index("s")

        n_real = 14 - c

        @pl.when(s < n_real)
        def _stage():
            krow = c * ROWS + s
            nfull = NP // PT
            bufs = (gb0, gb1)
            h = pltpu.async_copy(yt_hbm.at[pl.ds(krow * NP, PT)], gb0, sem_i)
            for ci in range(nfull):
                h.wait()
                if ci + 1 < nfull:
                    h = pltpu.async_copy(
                        yt_hbm.at[pl.ds(krow * NP + (ci + 1) * PT, PT)],
                        bufs[(ci + 1) % 2], sem_i)
                pltpu.sync_copy(bufs[ci % 2],
                                ytsh.at[pl.ds(s * NP + ci * PT, PT)])

        @pl.when((c == 1) & (s == ROWS - 1))
        def _zero_pad_row():
            def zv(j, carry):
                o = j * (16 * U)
                for u in range(U):
                    gb0[pl.ds(o + u * 16, 16)] = jnp.zeros((16,), f32)
                return carry

            lax.fori_loop(0, PT // (16 * U), zv, 0)
            for ci in range(NP // PT):
                pltpu.sync_copy(
                    gb0, ytsh.at[pl.ds((ROWS - 1) * NP + ci * PT, PT)])

        plsc.subcore_barrier()

        rbase = c * ROWS * NP + s * PT
        idxb = (idx0, idx1)
        gbufs = (gb0, gb1)

        pltpu.async_copy(gidx_hbm.at[pl.ds(rbase, PT)], idx0, sem_i).wait()
        gathers = [None] * ROWS
        gathers[0] = pltpu.async_copy(ytsh.at[idx0], gb0, sem_g)
        idx_pending = pltpu.async_copy(
            gidx_hbm.at[pl.ds(rbase + NP, PT)], idx1, sem_i)

        for g in range(ROWS):
            gathers[g].wait()
            if g + 1 < ROWS:
                idx_pending.wait()
                gathers[g + 1] = pltpu.async_copy(
                    ytsh.at[idxb[(g + 1) % 2]], gbufs[(g + 1) % 2], sem_g)
                if g + 2 < ROWS:
                    idx_pending = pltpu.async_copy(
                        gidx_hbm.at[pl.ds(rbase + (g + 2) * NP, PT)],
                        idxb[g % 2], sem_i)
            gb = gbufs[g % 2]

            def accum(j, carry, gb=gb, first=(g == 0)):
                o = j * (16 * U)
                for u in range(U):
                    oo = o + u * 16
                    if first:
                        acc[pl.ds(oo, 16)] = gb[pl.ds(oo, 16)]
                    else:
                        acc[pl.ds(oo, 16)] = acc[pl.ds(oo, 16)] + gb[pl.ds(oo, 16)]
                return carry

            lax.fori_loop(0, PT // (16 * U), accum, 0)

        @pl.when(c == 0)
        def _w0():
            pltpu.sync_copy(acc, s0_hbm.at[pl.ds(s * PT, PT)])

        @pl.when(c == 1)
        def _w1():
            pltpu.sync_copy(acc, s1_hbm.at[pl.ds(s * PT, PT)])

    return sc_gather


def kernel(x, neighbor_map, W):
    N, C = x.shape
    K = neighbor_map.shape[1]
    f32 = jnp.float32

    BC = 4096
    BA = 2048
    NB = (N + BA - 1) // BA
    NP = NB * BA

    Wk = W.reshape(K, C)
    xT = x.T
    PT = NP // 16
    offs = jnp.concatenate([
        jnp.arange(14, dtype=jnp.int32),
        jnp.arange(13, dtype=jnp.int32),
        jnp.full((1,), 13, dtype=jnp.int32)]) * NP
    nmT28 = jnp.concatenate([
        neighbor_map.T.astype(jnp.int32),
        jnp.zeros((1, N), jnp.int32)], axis=0)
    gidx = jnp.pad(nmT28 + offs[:, None], ((0, 0), (0, NP - N))).reshape(-1)

    yt = pl.pallas_call(
        _matmul_body,
        grid=(NB,),
        in_specs=[pl.BlockSpec((K, C), lambda i: (0, 0)),
                  pl.BlockSpec((C, BA), lambda i: (0, i))],
        out_specs=pl.BlockSpec((K, BA), lambda i: (0, i)),
        out_shape=jax.ShapeDtypeStruct((K, NP), f32),
    )(Wk, xT)

    sc = _make_sc_gather(K, N, NP)
    s0, s1 = sc(gidx, yt.reshape(-1))

    s0t = s0.reshape(1, NP)
    s1t = s1.reshape(1, NP)
    BB = 2048
    gb = (N + BB - 1) // BB
    outT = pl.pallas_call(
        _gate_body,
        grid=(gb,),
        in_specs=[pl.BlockSpec((C, BB), lambda i: (0, i)),
                  pl.BlockSpec((1, BB), lambda i: (0, i)),
                  pl.BlockSpec((1, BB), lambda i: (0, i))],
        out_specs=pl.BlockSpec((C, BB), lambda i: (0, i)),
        out_shape=jax.ShapeDtypeStruct((C, N), f32),
    )(xT, s0t, s1t)
    return outT.T


# speedup vs baseline: 3.9273x; 1.2711x over previous
import functools

import jax
import jax.numpy as jnp
from jax import lax
from jax.experimental import pallas as pl
from jax.experimental.pallas import tpu as pltpu
from jax.experimental.pallas import tpu_sc as plsc


def _matmul_body(w_ref, xt_ref, o_ref):
    o_ref[...] = jnp.dot(w_ref[...], xt_ref[...],
                         preferred_element_type=jnp.float32)


def _matmul_flat_body(w_ref, xt_ref, o_ref):
    k = pl.program_id(1)
    o_ref[...] = jnp.dot(w_ref[pl.ds(k, 1), :], xt_ref[...],
                         preferred_element_type=jnp.float32)


def _gate_body(xt_ref, a_ref, b_ref, o_ref):
    att = jax.nn.sigmoid(a_ref[...] + b_ref[...])
    o_ref[...] = xt_ref[...] * att


def _make_sc_gather(K, N, NP, R0):
    f32 = jnp.float32
    PT = NP // 16
    R1 = K - R0
    U = 8

    mesh = plsc.VectorSubcoreMesh(core_axis_name="c", subcore_axis_name="s")

    @functools.partial(
        pl.kernel,
        out_type=[jax.ShapeDtypeStruct((NP,), f32),
                  jax.ShapeDtypeStruct((NP,), f32)],
        mesh=mesh,
        compiler_params=pltpu.CompilerParams(needs_layout_passes=False),
        scratch_types=[
            pltpu.VMEM((PT,), jnp.int32),
            pltpu.VMEM((PT,), jnp.int32),
            pltpu.VMEM((PT,), f32),
            pltpu.VMEM((PT,), f32),
            pltpu.VMEM((PT,), f32),
            pltpu.VMEM_SHARED((R0 * NP,), f32),
            pltpu.SemaphoreType.DMA,
            pltpu.SemaphoreType.DMA,
        ],
    )
    def sc_gather(gidx_hbm, yt_hbm, s0_hbm, s1_hbm,
                  idx0, idx1, gb0, gb1, acc, ytsh, sem_i, sem_g):
        c = lax.axis_index("c")
        s = lax.axis_index("s")

        n_real = lax.select(c == 0, R0, R1)

        @pl.when(s < n_real)
        def _stage():
            krow = c * R0 + s
            nfull = NP // PT
            bufs = (gb0, gb1)
            h = pltpu.async_copy(yt_hbm.at[pl.ds(krow * NP, PT)], gb0, sem_i)
            for ci in range(nfull):
                h.wait()
                if ci + 1 < nfull:
                    h = pltpu.async_copy(
                        yt_hbm.at[pl.ds(krow * NP + (ci + 1) * PT, PT)],
                        bufs[(ci + 1) % 2], sem_i)
                pltpu.sync_copy(bufs[ci % 2],
                                ytsh.at[pl.ds(s * NP + ci * PT, PT)])

        plsc.subcore_barrier()

        idxb = (idx0, idx1)
        gbufs = (gb0, gb1)

        def pipe(nrows, row0, out_hbm):
            rbase = row0 * NP + s * PT
            pltpu.async_copy(gidx_hbm.at[pl.ds(rbase, PT)], idx0,
                             sem_i).wait()
            gathers = [None] * nrows
            gathers[0] = pltpu.async_copy(ytsh.at[idx0], gb0, sem_g)
            idx_pending = pltpu.async_copy(
                gidx_hbm.at[pl.ds(rbase + NP, PT)], idx1, sem_i)

            for g in range(nrows):
                gathers[g].wait()
                if g + 1 < nrows:
                    idx_pending.wait()
                    gathers[g + 1] = pltpu.async_copy(
                        ytsh.at[idxb[(g + 1) % 2]], gbufs[(g + 1) % 2], sem_g)
                    if g + 2 < nrows:
                        idx_pending = pltpu.async_copy(
                            gidx_hbm.at[pl.ds(rbase + (g + 2) * NP, PT)],
                            idxb[g % 2], sem_i)
                gb = gbufs[g % 2]

                def accum(j, carry, gb=gb, first=(g == 0)):
                    o = j * (16 * U)
                    for u in range(U):
                        oo = o + u * 16
                        if first:
                            acc[pl.ds(oo, 16)] = gb[pl.ds(oo, 16)]
                        else:
                            acc[pl.ds(oo, 16)] = (acc[pl.ds(oo, 16)]
                                                  + gb[pl.ds(oo, 16)])
                    return carry

                lax.fori_loop(0, PT // (16 * U), accum, 0)

            pltpu.sync_copy(acc, out_hbm.at[pl.ds(s * PT, PT)])

        @pl.when(c == 0)
        def _pipe0():
            pipe(R0, 0, s0_hbm)

        @pl.when(c == 1)
        def _pipe1():
            pipe(R1, R0, s1_hbm)

    return sc_gather


def kernel(x, neighbor_map, W):
    N, C = x.shape
    K = neighbor_map.shape[1]
    f32 = jnp.float32

    BC = 4096
    BA = 2048
    NB = (N + BA - 1) // BA
    NP = NB * BA

    Wk = W.reshape(K, C)
    xT = x.T
    PT = NP // 16
    R0 = 15
    offs = jnp.concatenate([
        jnp.arange(R0, dtype=jnp.int32),
        jnp.arange(K - R0, dtype=jnp.int32)]) * NP
    gidx = jnp.pad(neighbor_map.T.astype(jnp.int32) + offs[:, None],
                   ((0, 0), (0, NP - N))).reshape(-1)

    yt = pl.pallas_call(
        _matmul_body,
        grid=(NB,),
        in_specs=[pl.BlockSpec((K, C), lambda i: (0, 0)),
                  pl.BlockSpec((C, BA), lambda i: (0, i))],
        out_specs=pl.BlockSpec((K, BA), lambda i: (0, i)),
        out_shape=jax.ShapeDtypeStruct((K, NP), f32),
    )(Wk, xT)

    sc = _make_sc_gather(K, N, NP, R0)
    s0, s1 = sc(gidx, yt.reshape(-1))

    s0t = s0.reshape(1, NP)
    s1t = s1.reshape(1, NP)
    BB = 2048
    gb = (N + BB - 1) // BB
    outT = pl.pallas_call(
        _gate_body,
        grid=(gb,),
        in_specs=[pl.BlockSpec((C, BB), lambda i: (0, i)),
                  pl.BlockSpec((1, BB), lambda i: (0, i)),
                  pl.BlockSpec((1, BB), lambda i: (0, i))],
        out_specs=pl.BlockSpec((C, BB), lambda i: (0, i)),
        out_shape=jax.ShapeDtypeStruct((C, N), f32),
    )(xT, s0t, s1t)
    return outT.T


# speedup vs baseline: 3.9814x; 1.0138x over previous
import functools

import jax
import jax.numpy as jnp
from jax import lax
from jax.experimental import pallas as pl
from jax.experimental.pallas import tpu as pltpu
from jax.experimental.pallas import tpu_sc as plsc


def _matmul_body(w_ref, xt_ref, o_ref):
    o_ref[...] = jnp.dot(w_ref[...], xt_ref[...],
                         preferred_element_type=jnp.float32)


def _matmul_flat_body(w_ref, xt_ref, o_ref):
    k = pl.program_id(1)
    o_ref[...] = jnp.dot(w_ref[pl.ds(k, 1), :], xt_ref[...],
                         preferred_element_type=jnp.float32)


def _gate_body(xt_ref, a_ref, b_ref, o_ref):
    att = jax.nn.sigmoid(a_ref[...] + b_ref[...])
    o_ref[...] = xt_ref[...] * att


def _make_sc_gather(K, N, NP, R0):
    f32 = jnp.float32
    PT = NP // 16
    R1 = K - R0
    U = 8

    mesh = plsc.VectorSubcoreMesh(core_axis_name="c", subcore_axis_name="s")

    @functools.partial(
        pl.kernel,
        out_type=[jax.ShapeDtypeStruct((NP,), f32),
                  jax.ShapeDtypeStruct((NP,), f32)],
        mesh=mesh,
        compiler_params=pltpu.CompilerParams(needs_layout_passes=False),
        scratch_types=[
            pltpu.VMEM((PT,), jnp.int32),
            pltpu.VMEM((PT,), jnp.int32),
            pltpu.VMEM((PT,), f32),
            pltpu.VMEM((PT,), f32),
            pltpu.VMEM((PT,), f32),
            pltpu.VMEM_SHARED((R0 * NP,), f32),
            pltpu.SemaphoreType.DMA,
            pltpu.SemaphoreType.DMA,
        ],
    )
    def sc_gather(gidx_hbm, yt_hbm, s0_hbm, s1_hbm,
                  idx0, idx1, gb0, gb1, acc, ytsh, sem_i, sem_g):
        c = lax.axis_index("c")
        s = lax.axis_index("s")

        n_real = lax.select(c == 0, R0, R1)

        @pl.when(s < n_real)
        def _stage():
            krow = c * R0 + s
            nfull = NP // PT
            bufs = (gb0, gb1)
            h = pltpu.async_copy(yt_hbm.at[pl.ds(krow * NP, PT)], gb0, sem_i)
            for ci in range(nfull):
                h.wait()
                if ci + 1 < nfull:
                    h = pltpu.async_copy(
                        yt_hbm.at[pl.ds(krow * NP + (ci + 1) * PT, PT)],
                        bufs[(ci + 1) % 2], sem_i)
                pltpu.sync_copy(bufs[ci % 2],
                                ytsh.at[pl.ds(s * NP + ci * PT, PT)])

        plsc.subcore_barrier()

        idxb = (idx0, idx1)
        gbufs = (gb0, gb1)

        def pipe(nrows, row0, out_hbm):
            rbase = row0 * NP + s * PT
            pltpu.async_copy(gidx_hbm.at[pl.ds(rbase, PT)], idx0,
                             sem_i).wait()
            gathers = [None] * nrows
            gathers[0] = pltpu.async_copy(ytsh.at[idx0], gb0, sem_g)
            idx_pending = pltpu.async_copy(
                gidx_hbm.at[pl.ds(rbase + NP, PT)], idx1, sem_i)

            for g in range(nrows):
                gathers[g].wait()
                if g + 1 < nrows:
                    idx_pending.wait()
                    gathers[g + 1] = pltpu.async_copy(
                        ytsh.at[idxb[(g + 1) % 2]], gbufs[(g + 1) % 2], sem_g)
                    if g + 2 < nrows:
                        idx_pending = pltpu.async_copy(
                            gidx_hbm.at[pl.ds(rbase + (g + 2) * NP, PT)],
                            idxb[g % 2], sem_i)
                gb = gbufs[g % 2]

                def accum(j, carry, gb=gb, first=(g == 0)):
                    o = j * (16 * U)
                    for u in range(U):
                        oo = o + u * 16
                        if first:
                            acc[pl.ds(oo, 16)] = gb[pl.ds(oo, 16)]
                        else:
                            acc[pl.ds(oo, 16)] = (acc[pl.ds(oo, 16)]
                                                  + gb[pl.ds(oo, 16)])
                    return carry

                lax.fori_loop(0, PT // (16 * U), accum, 0)

            pltpu.sync_copy(acc, out_hbm.at[pl.ds(s * PT, PT)])

        @pl.when(c == 0)
        def _pipe0():
            pipe(R0, 0, s0_hbm)

        @pl.when(c == 1)
        def _pipe1():
            pipe(R1, R0, s1_hbm)

    return sc_gather


def kernel(x, neighbor_map, W):
    N, C = x.shape
    K = neighbor_map.shape[1]
    f32 = jnp.float32

    BC = 4096
    BA = 2048
    NB = (N + BA - 1) // BA
    NP = NB * BA

    Wk = W.reshape(K, C)
    xT = x.T
    PT = NP // 16
    R0 = 14
    offs = jnp.concatenate([
        jnp.arange(R0, dtype=jnp.int32),
        jnp.arange(K - R0, dtype=jnp.int32)]) * NP
    gidx = jnp.pad(neighbor_map.T.astype(jnp.int32) + offs[:, None],
                   ((0, 0), (0, NP - N))).reshape(-1)

    yt = pl.pallas_call(
        _matmul_body,
        grid=(NB,),
        in_specs=[pl.BlockSpec((K, C), lambda i: (0, 0)),
                  pl.BlockSpec((C, BA), lambda i: (0, i))],
        out_specs=pl.BlockSpec((K, BA), lambda i: (0, i)),
        out_shape=jax.ShapeDtypeStruct((K, NP), f32),
    )(Wk, xT)

    sc = _make_sc_gather(K, N, NP, R0)
    s0, s1 = sc(gidx, yt.reshape(-1))

    s0t = s0.reshape(1, NP)
    s1t = s1.reshape(1, NP)
    BB = 2048
    gb = (N + BB - 1) // BB
    outT = pl.pallas_call(
        _gate_body,
        grid=(gb,),
        in_specs=[pl.BlockSpec((C, BB), lambda i: (0, i)),
                  pl.BlockSpec((1, BB), lambda i: (0, i)),
                  pl.BlockSpec((1, BB), lambda i: (0, i))],
        out_specs=pl.BlockSpec((C, BB), lambda i: (0, i)),
        out_shape=jax.ShapeDtypeStruct((C, N), f32),
    )(xT, s0t, s1t)
    return outT.T


# speedup vs baseline: 3.9815x; 1.0000x over previous
import functools

import jax
import jax.numpy as jnp
from jax import lax
from jax.experimental import pallas as pl
from jax.experimental.pallas import tpu as pltpu
from jax.experimental.pallas import tpu_sc as plsc


def _matmul_body(w_ref, xt_ref, o_ref):
    o_ref[...] = jnp.dot(w_ref[...], xt_ref[...],
                         preferred_element_type=jnp.float32)


def _gate_body(xt_ref, a_ref, b_ref, o_ref):
    att = jax.nn.sigmoid(a_ref[...] + b_ref[...])
    o_ref[...] = xt_ref[...] * att


def _make_sc_gather(K, N, NP, R0):
    f32 = jnp.float32
    PT = NP // 16
    R1 = K - R0
    U = 8

    mesh = plsc.VectorSubcoreMesh(core_axis_name="c", subcore_axis_name="s")

    @functools.partial(
        pl.kernel,
        out_type=[jax.ShapeDtypeStruct((NP,), f32),
                  jax.ShapeDtypeStruct((NP,), f32)],
        mesh=mesh,
        compiler_params=pltpu.CompilerParams(needs_layout_passes=False),
        scratch_types=[
            pltpu.VMEM((PT,), jnp.int32),
            pltpu.VMEM((PT,), jnp.int32),
            pltpu.VMEM((PT,), f32),
            pltpu.VMEM((PT,), f32),
            pltpu.VMEM((PT,), f32),
            pltpu.VMEM_SHARED((R0 * NP,), f32),
            pltpu.SemaphoreType.DMA,
            pltpu.SemaphoreType.DMA,
        ],
    )
    def sc_gather(gidx_hbm, yt_hbm, s0_hbm, s1_hbm,
                  idx0, idx1, gb0, gb1, acc, ytsh, sem_i, sem_g):
        c = lax.axis_index("c")
        s = lax.axis_index("s")

        n_real = lax.select(c == 0, R0, R1)

        @pl.when(s < n_real)
        def _stage():
            krow = c * R0 + s
            nfull = NP // PT
            bufs = (gb0, gb1)
            h = pltpu.async_copy(yt_hbm.at[pl.ds(krow * NP, PT)], gb0, sem_i)
            for ci in range(nfull):
                h.wait()
                if ci + 1 < nfull:
                    h = pltpu.async_copy(
                        yt_hbm.at[pl.ds(krow * NP + (ci + 1) * PT, PT)],
                        bufs[(ci + 1) % 2], sem_i)
                pltpu.sync_copy(bufs[ci % 2],
                                ytsh.at[pl.ds(s * NP + ci * PT, PT)])

        plsc.subcore_barrier()

        idxb = (idx0, idx1)
        gbufs = (gb0, gb1)

        def pipe(nrows, row0, out_hbm):
            rbase = row0 * NP + s * PT
            pltpu.async_copy(gidx_hbm.at[pl.ds(rbase, PT)], idx0,
                             sem_i).wait()
            gathers = [None] * nrows
            gathers[0] = pltpu.async_copy(ytsh.at[idx0], gb0, sem_g)
            idx_pending = pltpu.async_copy(
                gidx_hbm.at[pl.ds(rbase + NP, PT)], idx1, sem_i)

            for g in range(nrows):
                gathers[g].wait()
                if g + 1 < nrows:
                    idx_pending.wait()
                    gathers[g + 1] = pltpu.async_copy(
                        ytsh.at[idxb[(g + 1) % 2]], gbufs[(g + 1) % 2], sem_g)
                    if g + 2 < nrows:
                        idx_pending = pltpu.async_copy(
                            gidx_hbm.at[pl.ds(rbase + (g + 2) * NP, PT)],
                            idxb[g % 2], sem_i)
                gb = gbufs[g % 2]

                def accum(j, carry, gb=gb, first=(g == 0)):
                    o = j * (16 * U)
                    for u in range(U):
                        oo = o + u * 16
                        if first:
                            acc[pl.ds(oo, 16)] = gb[pl.ds(oo, 16)]
                        else:
                            acc[pl.ds(oo, 16)] = (acc[pl.ds(oo, 16)]
                                                  + gb[pl.ds(oo, 16)])
                    return carry

                lax.fori_loop(0, PT // (16 * U), accum, 0)

            pltpu.sync_copy(acc, out_hbm.at[pl.ds(s * PT, PT)])

        @pl.when(c == 0)
        def _pipe0():
            pipe(R0, 0, s0_hbm)

        @pl.when(c == 1)
        def _pipe1():
            pipe(R1, R0, s1_hbm)

    return sc_gather


def kernel(x, neighbor_map, W):
    N, C = x.shape
    K = neighbor_map.shape[1]
    f32 = jnp.float32

    BC = 4096
    BA = 2048
    NB = (N + BA - 1) // BA
    NP = NB * BA

    Wk = W.reshape(K, C)
    xT = x.T
    PT = NP // 16
    R0 = 14
    offs = jnp.concatenate([
        jnp.arange(R0, dtype=jnp.int32),
        jnp.arange(K - R0, dtype=jnp.int32)]) * NP
    gidx = jnp.pad(neighbor_map.T.astype(jnp.int32) + offs[:, None],
                   ((0, 0), (0, NP - N))).reshape(-1)

    yt = pl.pallas_call(
        _matmul_body,
        grid=(NB,),
        in_specs=[pl.BlockSpec((K, C), lambda i: (0, 0)),
                  pl.BlockSpec((C, BA), lambda i: (0, i))],
        out_specs=pl.BlockSpec((K, BA), lambda i: (0, i)),
        out_shape=jax.ShapeDtypeStruct((K, NP), f32),
    )(Wk, xT)

    sc = _make_sc_gather(K, N, NP, R0)
    s0, s1 = sc(gidx, yt.reshape(-1))

    s0t = s0.reshape(1, NP)
    s1t = s1.reshape(1, NP)
    BB = 2048
    gb = (N + BB - 1) // BB
    outT = pl.pallas_call(
        _gate_body,
        grid=(gb,),
        in_specs=[pl.BlockSpec((C, BB), lambda i: (0, i)),
                  pl.BlockSpec((1, BB), lambda i: (0, i)),
                  pl.BlockSpec((1, BB), lambda i: (0, i))],
        out_specs=pl.BlockSpec((C, BB), lambda i: (0, i)),
        out_shape=jax.ShapeDtypeStruct((C, N), f32),
    )(xT, s0t, s1t)
    return outT.T
